# baseline (speedup 1.0000x reference)
import functools

import jax
import jax.numpy as jnp
from jax import lax
from jax.experimental import pallas as pl
from jax.experimental.pallas import tpu as pltpu
from jax.experimental.pallas import tpu_sc as plsc

_B = 2
_T = 4096
_H = 4096
_K = 8
_SLOTS = 64
_TB = 1024
_NT = _T // _TB

_NC = 2
_NS = 16
_NW = _NC * _NS


def _gate_body(w_ref, b_ref, x_ref, scores_ref, idx_ref, cv_ref, ci_ref):
    ti = pl.program_id(0)
    bi = pl.program_id(1)
    x = x_ref[0]
    w = w_ref[...]
    logits = lax.dot_general(
        w, x, (((1,), (1,)), ((), ())), preferred_element_type=jnp.float32
    )
    logits = logits + b_ref[0, 0]
    scores_ref[pl.ds(bi, 1), :] = jax.nn.sigmoid(logits)

    big = jnp.int32(_T)
    neg = jnp.float32(-jnp.inf)

    vals = logits
    pos = lax.broadcasted_iota(jnp.int32, (1, _TB), 1)
    for j in range(_K):
        m = jnp.max(vals)
        ij = jnp.min(jnp.where(vals == m, pos, big))
        cv_ref[bi, ti * _K + j] = m
        ci_ref[bi, ti * _K + j] = ti * _TB + ij
        vals = jnp.where(pos == ij, neg, vals)

    @pl.when(ti == _NT - 1)
    def _():
        nc = _NT * _K
        col = lax.broadcasted_iota(jnp.int32, (1, nc), 1)
        mv = jnp.full((1, nc), neg, jnp.float32)
        mi = jnp.zeros((1, nc), jnp.int32)
        for c in range(nc):
            mv = jnp.where(col == c, cv_ref[bi, c], mv)
            mi = jnp.where(col == c, ci_ref[bi, c], mi)
        for j in range(_K):
            m = jnp.max(mv)
            ij = jnp.min(jnp.where(mv == m, mi, big))
            idx_ref[(bi * _K + j) * 8] = bi * _T + ij
            mv = jnp.where(jnp.logical_and(mv == m, mi == ij), neg, mv)


def _gate(enc, w1h, b2d):
    return pl.pallas_call(
        _gate_body,
        grid=(_NT, _B),
        in_specs=[
            pl.BlockSpec((1, _H), lambda t, b: (0, 0)),
            pl.BlockSpec(memory_space=pltpu.SMEM),
            pl.BlockSpec((1, _TB, _H), lambda t, b: (b, t, 0)),
        ],
        out_specs=[
            pl.BlockSpec((_B, _TB), lambda t, b: (0, t)),
            pl.BlockSpec(memory_space=pltpu.SMEM),
        ],
        out_shape=[
            jax.ShapeDtypeStruct((_B, _T), jnp.float32),
            jax.ShapeDtypeStruct((_B * _K * 8,), jnp.int32),
        ],
        scratch_shapes=[
            pltpu.SMEM((_B, _NT * _K), jnp.float32),
            pltpu.SMEM((_B, _NT * _K), jnp.int32),
        ],
    )(w1h, b2d, enc)


def _sc_zero_memory():
    mesh = plsc.VectorSubcoreMesh(core_axis_name="c", subcore_axis_name="s")

    @functools.partial(
        pl.kernel,
        mesh=mesh,
        out_type=jax.ShapeDtypeStruct((_B * _SLOTS, _H), jnp.float32),
        scratch_types=[
            pltpu.VMEM((_H,), jnp.float32),
            pltpu.SemaphoreType.DMA,
        ],
    )
    def k(out_hbm, zrow_v, sem):
        cid = lax.axis_index("c")
        sid = lax.axis_index("s")
        wid = sid * _NC + cid
        z16 = jnp.zeros((16,), jnp.float32)

        @pl.loop(0, _H, step=16)
        def _(i):
            zrow_v[pl.ds(i, 16)] = z16

        copies = [
            pltpu.async_copy(zrow_v, out_hbm.at[wid * 4 + r], sem)
            for r in range(4)
        ]
        for c in copies:
            c.wait()

    return k()


def _sc_gather_memory(enc2d, gidx, mem_ref):
    mesh = plsc.VectorSubcoreMesh(core_axis_name="c", subcore_axis_name="s")

    @functools.partial(
        pl.kernel,
        mesh=mesh,
        scratch_types=[
            pltpu.VMEM((_B * _K * 8,), jnp.int32),
            pltpu.VMEM((1, _H), jnp.float32),
            pltpu.SemaphoreType.DMA,
        ],
    )
    def k(enc_hbm, gidx_hbm, out_hbm, idx_v, row_v, sem):
        cid = lax.axis_index("c")
        sid = lax.axis_index("s")
        wid = sid * _NC + cid

        @pl.when(wid < _B * _K)
        def _():
            pltpu.sync_copy(gidx_hbm, idx_v)
            pltpu.async_copy(
                enc_hbm.at[idx_v.at[pl.ds(wid * 8, 1)]], row_v, sem
            ).wait()
            dst = (wid // _K) * _SLOTS + lax.rem(wid, _K)
            pltpu.sync_copy(row_v, out_hbm.at[pl.ds(dst, 1)])

    k(enc2d, gidx, mem_ref)


def kernel(enc_hidden, W, b):
    w1h = W.reshape(1, _H)
    b2d = b.reshape(1, 1)
    gate_scores, gidx = _gate(enc_hidden, w1h, b2d)
    enc2d = enc_hidden.reshape(_B * _T, _H)
    mem0 = _sc_zero_memory()
    mem_ref = jax.new_ref(mem0)
    _sc_gather_memory(enc2d, gidx, mem_ref)
    memory = jax.freeze(mem_ref).reshape(_B, _SLOTS, _H)
    return (memory, gate_scores)


# speedup vs baseline: 1.1061x; 1.1061x over previous
import functools

import jax
import jax.numpy as jnp
from jax import lax
from jax.experimental import pallas as pl
from jax.experimental.pallas import tpu as pltpu
from jax.experimental.pallas import tpu_sc as plsc

_B = 2
_T = 4096
_H = 4096
_K = 8
_SLOTS = 64
_TB = 1024
_NT = _T // _TB

_NC = 2
_NS = 16
_NW = _NC * _NS


def _gate_body(w_ref, b_ref, x_ref, scores_ref, idx_ref, acc_ref):
    ti = pl.program_id(0)
    bi = pl.program_id(1)
    x = x_ref[0]
    w = w_ref[...]
    logits = lax.dot_general(
        w, x, (((1,), (1,)), ((), ())), preferred_element_type=jnp.float32
    )
    logits = logits + b_ref[0, 0]
    scores_ref[pl.ds(bi, 1), :] = jax.nn.sigmoid(logits)
    acc_ref[pl.ds(bi, 1), pl.ds(ti, 1), :] = logits[None]

    @pl.when(ti == _NT - 1)
    def _():
        vals = acc_ref[bi]
        rows = lax.broadcasted_iota(jnp.int32, (_NT, _TB), 0)
        cols = lax.broadcasted_iota(jnp.int32, (_NT, _TB), 1)
        gpos = rows * _TB + cols
        big = jnp.int32(_T)
        neg = jnp.float32(-jnp.inf)
        for j in range(_K):
            m = jnp.max(vals)
            ij = jnp.min(jnp.where(vals == m, gpos, big))
            idx_ref[(bi * _K + j) * 8] = bi * _T + ij
            vals = jnp.where(gpos == ij, neg, vals)


def _gate(enc, w1h, b2d):
    return pl.pallas_call(
        _gate_body,
        grid=(_NT, _B),
        in_specs=[
            pl.BlockSpec((1, _H), lambda t, b: (0, 0)),
            pl.BlockSpec(memory_space=pltpu.SMEM),
            pl.BlockSpec((1, _TB, _H), lambda t, b: (b, t, 0)),
        ],
        out_specs=[
            pl.BlockSpec((_B, _TB), lambda t, b: (0, t)),
            pl.BlockSpec(memory_space=pltpu.SMEM),
        ],
        out_shape=[
            jax.ShapeDtypeStruct((_B, _T), jnp.float32),
            jax.ShapeDtypeStruct((_B * _K * 8,), jnp.int32),
        ],
        scratch_shapes=[pltpu.VMEM((_B, _NT, _TB), jnp.float32)],
    )(w1h, b2d, enc)


def _sc_zero_memory():
    mesh = plsc.VectorSubcoreMesh(core_axis_name="c", subcore_axis_name="s")

    @functools.partial(
        pl.kernel,
        mesh=mesh,
        out_type=jax.ShapeDtypeStruct((_B * _SLOTS, _H), jnp.float32),
        scratch_types=[
            pltpu.VMEM((_H,), jnp.float32),
            pltpu.SemaphoreType.DMA,
        ],
    )
    def k(out_hbm, zrow_v, sem):
        cid = lax.axis_index("c")
        sid = lax.axis_index("s")
        wid = sid * _NC + cid
        z16 = jnp.zeros((16,), jnp.float32)

        @pl.loop(0, _H, step=16)
        def _(i):
            zrow_v[pl.ds(i, 16)] = z16

        copies = [
            pltpu.async_copy(zrow_v, out_hbm.at[wid * 4 + r], sem)
            for r in range(4)
        ]
        for c in copies:
            c.wait()

    return k()


def _sc_gather_memory(enc2d, gidx, mem_ref):
    mesh = plsc.VectorSubcoreMesh(core_axis_name="c", subcore_axis_name="s")

    @functools.partial(
        pl.kernel,
        mesh=mesh,
        scratch_types=[
            pltpu.VMEM((_B * _K * 8,), jnp.int32),
            pltpu.VMEM((1, _H), jnp.float32),
            pltpu.SemaphoreType.DMA,
        ],
    )
    def k(enc_hbm, gidx_hbm, out_hbm, idx_v, row_v, sem):
        cid = lax.axis_index("c")
        sid = lax.axis_index("s")
        wid = sid * _NC + cid

        @pl.when(wid < _B * _K)
        def _():
            pltpu.sync_copy(gidx_hbm, idx_v)
            pltpu.async_copy(
                enc_hbm.at[idx_v.at[pl.ds(wid * 8, 1)]], row_v, sem
            ).wait()
            dst = (wid // _K) * _SLOTS + lax.rem(wid, _K)
            pltpu.sync_copy(row_v, out_hbm.at[pl.ds(dst, 1)])

    k(enc2d, gidx, mem_ref)


def kernel(enc_hidden, W, b):
    w1h = W.reshape(1, _H)
    b2d = b.reshape(1, 1)
    gate_scores, gidx = _gate(enc_hidden, w1h, b2d)
    enc2d = enc_hidden.reshape(_B * _T, _H)
    mem0 = _sc_zero_memory()
    mem_ref = jax.new_ref(mem0)
    _sc_gather_memory(enc2d, gidx, mem_ref)
    memory = jax.freeze(mem_ref).reshape(_B, _SLOTS, _H)
    return (memory, gate_scores)


# speedup vs baseline: 1.1093x; 1.0029x over previous
import functools

import jax
import jax.numpy as jnp
from jax import lax
from jax.experimental import pallas as pl
from jax.experimental.pallas import tpu as pltpu
from jax.experimental.pallas import tpu_sc as plsc

_B = 2
_T = 4096
_H = 4096
_K = 8
_SLOTS = 64
_TB = 1024
_NT = _T // _TB

_NC = 2
_NS = 16
_NW = _NC * _NS


def _gate_body(w_ref, b_ref, x_ref, scores_ref, idx_ref, acc_ref):
    ti = pl.program_id(0)
    bi = pl.program_id(1)
    w = w_ref[...]
    logits = lax.dot_general(
        w, x_ref[0], (((1,), (1,)), ((), ())),
        preferred_element_type=jnp.float32,
    ) + b_ref[0, 0]
    scores_ref[pl.ds(bi, 1), :] = jax.nn.sigmoid(logits)
    sub = _TB // 128
    for r8 in range(sub):
        acc_ref[pl.ds(bi, 1), pl.ds(ti * sub + r8, 1), :] = (
            logits[:, r8 * 128:(r8 + 1) * 128][None]
        )

    @pl.when(ti == _NT - 1)
    def _():
        nr = _T // 128
        vals = acc_ref[bi]
        rows = lax.broadcasted_iota(jnp.int32, (nr, 128), 0)
        cols = lax.broadcasted_iota(jnp.int32, (nr, 128), 1)
        gpos = rows * 128 + cols
        big = jnp.int32(_T)
        neg = jnp.float32(-jnp.inf)
        for j in range(_K):
            m = jnp.max(vals)
            ij = jnp.min(jnp.where(vals == m, gpos, big))
            idx_ref[(bi * _K + j) * 8] = bi * _T + ij
            vals = jnp.where(gpos == ij, neg, vals)


def _gate(enc, w1h, b2d):
    return pl.pallas_call(
        _gate_body,
        grid=(_NT, _B),
        in_specs=[
            pl.BlockSpec((1, _H), lambda t, b: (0, 0)),
            pl.BlockSpec(memory_space=pltpu.SMEM),
            pl.BlockSpec((1, _TB, _H), lambda t, b: (b, t, 0)),
        ],
        out_specs=[
            pl.BlockSpec((_B, _TB), lambda t, b: (0, t)),
            pl.BlockSpec(memory_space=pltpu.SMEM),
        ],
        out_shape=[
            jax.ShapeDtypeStruct((_B, _T), jnp.float32),
            jax.ShapeDtypeStruct((_B * _K * 8,), jnp.int32),
        ],
        scratch_shapes=[pltpu.VMEM((_B, _T // 128, 128), jnp.float32)],
    )(w1h, b2d, enc)


def _sc_zero_memory():
    mesh = plsc.VectorSubcoreMesh(core_axis_name="c", subcore_axis_name="s")

    @functools.partial(
        pl.kernel,
        mesh=mesh,
        out_type=jax.ShapeDtypeStruct((_B * _SLOTS, _H), jnp.float32),
        scratch_types=[
            pltpu.VMEM((_H,), jnp.float32),
            pltpu.SemaphoreType.DMA,
        ],
    )
    def k(out_hbm, zrow_v, sem):
        cid = lax.axis_index("c")
        sid = lax.axis_index("s")
        wid = sid * _NC + cid
        z16 = jnp.zeros((16,), jnp.float32)

        @pl.loop(0, _H, step=16)
        def _(i):
            zrow_v[pl.ds(i, 16)] = z16

        copies = [
            pltpu.async_copy(zrow_v, out_hbm.at[wid * 4 + r], sem)
            for r in range(4)
        ]
        for c in copies:
            c.wait()

    return k()


def _sc_gather_memory(enc2d, gidx, mem_ref):
    mesh = plsc.VectorSubcoreMesh(core_axis_name="c", subcore_axis_name="s")

    @functools.partial(
        pl.kernel,
        mesh=mesh,
        scratch_types=[
            pltpu.VMEM((_B * _K * 8,), jnp.int32),
            pltpu.VMEM((1, _H), jnp.float32),
            pltpu.SemaphoreType.DMA,
        ],
    )
    def k(enc_hbm, gidx_hbm, out_hbm, idx_v, row_v, sem):
        cid = lax.axis_index("c")
        sid = lax.axis_index("s")
        wid = sid * _NC + cid

        @pl.when(wid < _B * _K)
        def _():
            pltpu.sync_copy(gidx_hbm, idx_v)
            pltpu.async_copy(
                enc_hbm.at[idx_v.at[pl.ds(wid * 8, 1)]], row_v, sem
            ).wait()
            dst = (wid // _K) * _SLOTS + lax.rem(wid, _K)
            pltpu.sync_copy(row_v, out_hbm.at[pl.ds(dst, 1)])

    k(enc2d, gidx, mem_ref)


def kernel(enc_hidden, W, b):
    w1h = W.reshape(1, _H)
    b2d = b.reshape(1, 1)
    gate_scores, gidx = _gate(enc_hidden, w1h, b2d)
    enc2d = enc_hidden.reshape(_B * _T, _H)
    mem0 = _sc_zero_memory()
    mem_ref = jax.new_ref(mem0)
    _sc_gather_memory(enc2d, gidx, mem_ref)
    memory = jax.freeze(mem_ref).reshape(_B, _SLOTS, _H)
    return (memory, gate_scores)
